# Initial kernel scaffold; baseline (speedup 1.0000x reference)
import jax
import jax.numpy as jnp
from jax.experimental import pallas as pl


def kernel(h, e, Wq, Wk, Wv, We, Oh_w, Oh_b, Oe_w, Oe_b, bn1h_g, bn1h_b, bn1e_g, bn1e_b, fh1_w, fh1_b, fh2_w, fh2_b, fe1_w, fe1_b, fe2_w, fe2_b, bn2h_g, bn2h_b, bn2e_g, bn2e_b, edge_index):
    raise NotImplementedError("write your pallas kernel here")


# speedup vs baseline: 19.9816x; 19.9816x over previous
import functools
import jax
import jax.numpy as jnp
from jax import lax
from jax.experimental import pallas as pl
from jax.experimental.pallas import tpu as pltpu
from jax.experimental.pallas import tpu_sc as plsc

N = 10000
E = 320000
D = 128
H = 8
DH = D // H

N_PAD = 10240
NTILES = 32
EPT = E // NTILES
C = 40
NCHUNK = EPT // C
ROWS_PER_TILE = N_PAD // 16
ZCHUNK = C

EBLK = 3200
EGRID = E // EBLK


def _qkv_body(h_ref, wq_ref, wk_ref, wv_ref, q_ref, k_ref, v_ref):
    hb = h_ref[...]
    q_ref[...] = jnp.dot(hb, wq_ref[...].T, preferred_element_type=jnp.float32)
    k_ref[...] = jnp.dot(hb, wk_ref[...].T, preferred_element_type=jnp.float32) * (1.0 / (DH ** 0.5))
    v_ref[...] = jnp.dot(hb, wv_ref[...].T, preferred_element_type=jnp.float32)


def _qkv(h, Wq, Wk, Wv):
    return pl.pallas_call(
        _qkv_body,
        out_shape=[jax.ShapeDtypeStruct((N, D), jnp.float32)] * 3,
    )(h, Wq, Wk, Wv)


def _pe_body(e_ref, we_ref, pe_ref):
    pe_ref[...] = jnp.dot(e_ref[...], we_ref[...].T, preferred_element_type=jnp.float32)


def _pe(e, We):
    return pl.pallas_call(
        _pe_body,
        grid=(EGRID,),
        in_specs=[
            pl.BlockSpec((EBLK, D), lambda i: (i, 0)),
            pl.BlockSpec((D, D), lambda i: (0, 0)),
        ],
        out_specs=pl.BlockSpec((EBLK, D), lambda i: (i, 0)),
        out_shape=jax.ShapeDtypeStruct((E, D), jnp.float32),
    )(e, We)


NZ = N_PAD // 8
ZROWS_PER_TILE = NZ // 16


def _edge_sc_body(kh_hbm, qh_hbm, vh_hbm, pe_hbm, src_hbm, dst_hbm,
                  eo_hbm, wv_hbm, z_hbm,
                  src_v, dst_v, dstp_v, zrow_v, k_v, q_v, v_v, pe_v, eo_v, wv_v, zp_v,
                  shared_wv, shared_zp, sem1, sem2, sem3):
    c = lax.axis_index("c")
    s = lax.axis_index("s")
    wid = c * 16 + s
    tile_base = wid * EPT
    lanes = lax.iota(jnp.int32, 16)

    def _zero_row(i, _):
        for hh in range(H):
            wv_v[i, hh * DH:(hh + 1) * DH] = jnp.zeros((16,), jnp.float32)
        return 0
    lax.fori_loop(0, C, _zero_row, 0)

    row0 = s * ROWS_PER_TILE
    def _zero_shared(j, _):
        r = row0 + j * ZCHUNK
        pltpu.sync_copy(wv_v, shared_wv.at[pl.ds(r, ZCHUNK)])
        return 0
    lax.fori_loop(0, ROWS_PER_TILE // ZCHUNK, _zero_shared, 0)

    zrow0 = s * ZROWS_PER_TILE
    def _zero_sharedz(j, _):
        r = zrow0 + j * ZCHUNK
        pltpu.sync_copy(wv_v, shared_zp.at[pl.ds(r, ZCHUNK)])
        return 0
    lax.fori_loop(0, ZROWS_PER_TILE // ZCHUNK, _zero_sharedz, 0)
    plsc.subcore_barrier()

    def _chunk(ci, _):
        base = tile_base + ci * C
        pltpu.sync_copy(src_hbm.at[pl.ds(base, C)], src_v)
        pltpu.sync_copy(dst_hbm.at[pl.ds(base, C)], dst_v)
        pltpu.sync_copy(dst_hbm.at[pl.ds(base, C)], dstp_v.at[pl.ds(0, C)])
        cp1 = pltpu.async_copy(kh_hbm.at[src_v], k_v, sem1)
        cp2 = pltpu.async_copy(qh_hbm.at[dst_v], q_v, sem2)
        cp3 = pltpu.async_copy(vh_hbm.at[src_v], v_v, sem3)
        pltpu.sync_copy(pe_hbm.at[pl.ds(base, C)], pe_v)
        for g in (0, 16, 24):
            dd = dst_v[pl.ds(g, 16)]
            zrow_v[pl.ds(g, 16)] = lax.shift_right_logical(dd, 3)
        cp1.wait()
        cp2.wait()
        cp3.wait()

        perms = [(lanes ^ kk).reshape(16, 1) for kk in (1, 2, 4, 8)]
        dnums = lax.GatherDimensionNumbers(
            offset_dims=(), collapsed_slice_dims=(0,), start_index_map=(0,))

        def _shuf(x, p):
            return lax.gather(x, p, dnums, (1,),
                              mode=lax.GatherScatterMode.PROMISE_IN_BOUNDS)

        def _edge(i, _):
            z_vec = jnp.zeros((16,), jnp.float32)
            for hh in range(H):
                sl = slice(hh * DH, (hh + 1) * DH)
                sc = k_v[i, sl] * q_v[i, sl] * pe_v[i, sl]
                eo_v[i, sl] = sc
                tot = sc
                for p in perms:
                    tot = tot + _shuf(tot, p)
                sv = jnp.exp(jnp.minimum(jnp.maximum(tot, -5.0), 5.0))
                wv_v[i, sl] = v_v[i, sl] * sv
                z_vec = jnp.where(lanes == hh, sv, z_vec)
                zp_v[i, sl] = jnp.zeros((16,), jnp.float32)
            dv = dstp_v[pl.ds(i, 16)]
            colbase = (dv[0] & 7) * DH
            zp_v[i, pl.ds(colbase, DH)] = z_vec
            return 0
        lax.fori_loop(0, C, _edge, 0)

        pltpu.sync_copy(eo_v, eo_hbm.at[pl.ds(base, C)])
        pltpu.sync_copy(wv_v, shared_wv.at[dst_v], add=True)
        pltpu.sync_copy(zp_v, shared_zp.at[zrow_v], add=True)
        return 0
    lax.fori_loop(0, NCHUNK, _chunk, 0)

    plsc.subcore_barrier()

    def _out(j, _):
        r = row0 + j * ZCHUNK
        pltpu.sync_copy(shared_wv.at[pl.ds(r, ZCHUNK)], wv_v)
        pltpu.sync_copy(wv_v, wv_hbm.at[c, pl.ds(r, ZCHUNK)])
        return 0
    lax.fori_loop(0, ROWS_PER_TILE // ZCHUNK, _out, 0)

    def _outz(j, _):
        r = zrow0 + j * ZCHUNK
        pltpu.sync_copy(shared_zp.at[pl.ds(r, ZCHUNK)], zp_v)
        pltpu.sync_copy(zp_v, z_hbm.at[c, pl.ds(r, ZCHUNK)])
        return 0
    lax.fori_loop(0, ZROWS_PER_TILE // ZCHUNK, _outz, 0)


def _edge_sc(kh, qh, vh, pe, src, dst):
    mesh = plsc.VectorSubcoreMesh(core_axis_name="c", subcore_axis_name="s")
    f = pl.kernel(
        _edge_sc_body,
        out_type=[
            jax.ShapeDtypeStruct((E, D), jnp.float32),
            jax.ShapeDtypeStruct((2, N_PAD, D), jnp.float32),
            jax.ShapeDtypeStruct((2, NZ, D), jnp.float32),
        ],
        mesh=mesh,
        scratch_types=[
            pltpu.VMEM((C,), jnp.int32),
            pltpu.VMEM((C,), jnp.int32),
            pltpu.VMEM((C + 16,), jnp.int32),
            pltpu.VMEM((C,), jnp.int32),
            pltpu.VMEM((C, D), jnp.float32),
            pltpu.VMEM((C, D), jnp.float32),
            pltpu.VMEM((C, D), jnp.float32),
            pltpu.VMEM((C, D), jnp.float32),
            pltpu.VMEM((C, D), jnp.float32),
            pltpu.VMEM((C, D), jnp.float32),
            pltpu.VMEM((C, D), jnp.float32),
            pltpu.VMEM_SHARED((N_PAD, D), jnp.float32),
            pltpu.VMEM_SHARED((NZ, D), jnp.float32),
            pltpu.SemaphoreType.DMA,
            pltpu.SemaphoreType.DMA,
            pltpu.SemaphoreType.DMA,
        ],
    )
    return f(kh, qh, vh, pe, src, dst)


def _node_body(wvp_ref, zp_ref, h_ref, ohw_ref, ohb_ref,
               g1_ref, b1_ref, f1w_ref, f1b_ref, f2w_ref, f2b_ref,
               g2_ref, b2_ref, out_ref):
    wv = wvp_ref[0, :N, :] + wvp_ref[1, :N, :]
    z8 = zp_ref[0, :N, :8] + zp_ref[1, :N, :8]
    rows = lax.broadcasted_iota(jnp.int32, (8, D), 0)
    cols = lax.broadcasted_iota(jnp.int32, (8, D), 1) // DH
    expand = (rows == cols).astype(jnp.float32)
    z_exp = jnp.dot(z8, expand, preferred_element_type=jnp.float32)
    h_attn = wv / (z_exp + 1e-6)
    h1 = jnp.dot(h_attn, ohw_ref[...].T, preferred_element_type=jnp.float32) + ohb_ref[...]
    h1 = h1 + h_ref[...]
    m = jnp.mean(h1, axis=0, keepdims=True)
    v = jnp.mean(h1 * h1, axis=0, keepdims=True) - m * m
    h1 = (h1 - m) * (g1_ref[...] / jnp.sqrt(v + 1e-5)) + b1_ref[...]
    t = jnp.maximum(jnp.dot(h1, f1w_ref[...].T, preferred_element_type=jnp.float32) + f1b_ref[...], 0.0)
    h2 = h1 + jnp.dot(t, f2w_ref[...].T, preferred_element_type=jnp.float32) + f2b_ref[...]
    m2 = jnp.mean(h2, axis=0, keepdims=True)
    v2 = jnp.mean(h2 * h2, axis=0, keepdims=True) - m2 * m2
    out_ref[...] = (h2 - m2) * (g2_ref[...] / jnp.sqrt(v2 + 1e-5)) + b2_ref[...]


def _node(wvp, zp, h, Oh_w, Oh_b, g1, b1, f1w, f1b, f2w, f2b, g2, b2):
    return pl.pallas_call(
        _node_body,
        out_shape=jax.ShapeDtypeStruct((N, D), jnp.float32),
    )(wvp, zp, h, Oh_w, Oh_b, g1, b1, f1w, f1b, f2w, f2b, g2, b2)


def _e1_body(eo_ref, e_ref, oew_ref, oeb_ref, out_ref, st_ref):
    x = jnp.dot(eo_ref[...], oew_ref[...].T, preferred_element_type=jnp.float32)
    x = x + oeb_ref[...] + e_ref[...]
    out_ref[...] = x
    i = pl.program_id(0)

    @pl.when(i == 0)
    def _():
        st_ref[...] = jnp.zeros_like(st_ref)

    s1 = jnp.sum(x, axis=0, keepdims=True)
    s2 = jnp.sum(x * x, axis=0, keepdims=True)
    st_ref[...] += jnp.concatenate([s1, s2], axis=0)


def _e1(e_out, e, Oe_w, Oe_b):
    return pl.pallas_call(
        _e1_body,
        grid=(EGRID,),
        in_specs=[
            pl.BlockSpec((EBLK, D), lambda i: (i, 0)),
            pl.BlockSpec((EBLK, D), lambda i: (i, 0)),
            pl.BlockSpec((D, D), lambda i: (0, 0)),
            pl.BlockSpec((1, D), lambda i: (0, 0)),
        ],
        out_specs=[
            pl.BlockSpec((EBLK, D), lambda i: (i, 0)),
            pl.BlockSpec((2, D), lambda i: (0, 0)),
        ],
        out_shape=[
            jax.ShapeDtypeStruct((E, D), jnp.float32),
            jax.ShapeDtypeStruct((2, D), jnp.float32),
        ],
    )(e_out, e, Oe_w, Oe_b)


def _e2_body(x_ref, st_ref, g_ref, b_ref, f1w_ref, f1b_ref, f2w_ref, f2b_ref,
             out_ref, st2_ref):
    m = st_ref[0:1, :] * (1.0 / E)
    v = st_ref[1:2, :] * (1.0 / E) - m * m
    a = g_ref[...] / jnp.sqrt(v + 1e-5)
    cc = b_ref[...] - m * a
    x = x_ref[...] * a + cc
    t = jnp.maximum(jnp.dot(x, f1w_ref[...].T, preferred_element_type=jnp.float32) + f1b_ref[...], 0.0)
    y = x + jnp.dot(t, f2w_ref[...].T, preferred_element_type=jnp.float32) + f2b_ref[...]
    out_ref[...] = y
    i = pl.program_id(0)

    @pl.when(i == 0)
    def _():
        st2_ref[...] = jnp.zeros_like(st2_ref)

    s1 = jnp.sum(y, axis=0, keepdims=True)
    s2 = jnp.sum(y * y, axis=0, keepdims=True)
    st2_ref[...] += jnp.concatenate([s1, s2], axis=0)


def _e2(e1_raw, st1, g, b, f1w, f1b, f2w, f2b):
    return pl.pallas_call(
        _e2_body,
        grid=(EGRID,),
        in_specs=[
            pl.BlockSpec((EBLK, D), lambda i: (i, 0)),
            pl.BlockSpec((2, D), lambda i: (0, 0)),
            pl.BlockSpec((1, D), lambda i: (0, 0)),
            pl.BlockSpec((1, D), lambda i: (0, 0)),
            pl.BlockSpec((2 * D, D), lambda i: (0, 0)),
            pl.BlockSpec((1, 2 * D), lambda i: (0, 0)),
            pl.BlockSpec((D, 2 * D), lambda i: (0, 0)),
            pl.BlockSpec((1, D), lambda i: (0, 0)),
        ],
        out_specs=[
            pl.BlockSpec((EBLK, D), lambda i: (i, 0)),
            pl.BlockSpec((2, D), lambda i: (0, 0)),
        ],
        out_shape=[
            jax.ShapeDtypeStruct((E, D), jnp.float32),
            jax.ShapeDtypeStruct((2, D), jnp.float32),
        ],
    )(e1_raw, st1, g, b, f1w, f1b, f2w, f2b)


def _e3_body(x_ref, st_ref, g_ref, b_ref, out_ref):
    m = st_ref[0:1, :] * (1.0 / E)
    v = st_ref[1:2, :] * (1.0 / E) - m * m
    a = g_ref[...] / jnp.sqrt(v + 1e-5)
    cc = b_ref[...] - m * a
    out_ref[...] = x_ref[...] * a + cc


def _e3(e2_raw, st2, g, b):
    return pl.pallas_call(
        _e3_body,
        grid=(EGRID,),
        in_specs=[
            pl.BlockSpec((EBLK, D), lambda i: (i, 0)),
            pl.BlockSpec((2, D), lambda i: (0, 0)),
            pl.BlockSpec((1, D), lambda i: (0, 0)),
            pl.BlockSpec((1, D), lambda i: (0, 0)),
        ],
        out_specs=pl.BlockSpec((EBLK, D), lambda i: (i, 0)),
        out_shape=jax.ShapeDtypeStruct((E, D), jnp.float32),
    )(e2_raw, st2, g, b)


@jax.jit
def kernel(h, e, Wq, Wk, Wv, We, Oh_w, Oh_b, Oe_w, Oe_b,
           bn1h_g, bn1h_b, bn1e_g, bn1e_b,
           fh1_w, fh1_b, fh2_w, fh2_b, fe1_w, fe1_b, fe2_w, fe2_b,
           bn2h_g, bn2h_b, bn2e_g, bn2e_b, edge_index):
    src = edge_index[0].astype(jnp.int32)
    dst = edge_index[1].astype(jnp.int32)

    qh, kh_s, vh = _qkv(h, Wq, Wk, Wv)
    pe = _pe(e, We)
    e_out, wvp, zp_packed = _edge_sc(kh_s, qh, vh, pe, src, dst)
    zp = zp_packed.reshape(2, N_PAD, 16)

    h2 = _node(wvp, zp, h, Oh_w, Oh_b.reshape(1, D),
               bn1h_g.reshape(1, D), bn1h_b.reshape(1, D),
               fh1_w, fh1_b.reshape(1, 2 * D), fh2_w, fh2_b.reshape(1, D),
               bn2h_g.reshape(1, D), bn2h_b.reshape(1, D))

    e1_raw, st1 = _e1(e_out, e, Oe_w, Oe_b.reshape(1, D))
    e2_raw, st2 = _e2(e1_raw, st1, bn1e_g.reshape(1, D), bn1e_b.reshape(1, D),
                      fe1_w, fe1_b.reshape(1, 2 * D), fe2_w, fe2_b.reshape(1, D))
    e2 = _e3(e2_raw, st2, bn2e_g.reshape(1, D), bn2e_b.reshape(1, D))
    return (h2, e2)


# speedup vs baseline: 22.3208x; 1.1171x over previous
import functools
import jax
import jax.numpy as jnp
from jax import lax
from jax.experimental import pallas as pl
from jax.experimental.pallas import tpu as pltpu
from jax.experimental.pallas import tpu_sc as plsc

N = 10000
E = 320000
D = 128
H = 8
DH = D // H

N_PAD = 10240
NTILES = 32
EPT = E // NTILES
C = 40
NCHUNK = EPT // C
ROWS_PER_TILE = N_PAD // 16
ZCHUNK = C

EBLK = 3200
EGRID = E // EBLK


def _qkv_body(h_ref, wq_ref, wk_ref, wv_ref, q_ref, k_ref, v_ref):
    hb = h_ref[...]
    q_ref[...] = jnp.dot(hb, wq_ref[...].T, preferred_element_type=jnp.float32)
    k_ref[...] = jnp.dot(hb, wk_ref[...].T, preferred_element_type=jnp.float32) * (1.0 / (DH ** 0.5))
    v_ref[...] = jnp.dot(hb, wv_ref[...].T, preferred_element_type=jnp.float32)


def _qkv(h, Wq, Wk, Wv):
    return pl.pallas_call(
        _qkv_body,
        out_shape=[jax.ShapeDtypeStruct((N, D), jnp.float32)] * 3,
    )(h, Wq, Wk, Wv)


def _pe_body(e_ref, we_ref, pe_ref):
    pe_ref[...] = jnp.dot(e_ref[...], we_ref[...].T, preferred_element_type=jnp.float32)


def _pe(e, We):
    return pl.pallas_call(
        _pe_body,
        grid=(EGRID,),
        in_specs=[
            pl.BlockSpec((EBLK, D), lambda i: (i, 0)),
            pl.BlockSpec((D, D), lambda i: (0, 0)),
        ],
        out_specs=pl.BlockSpec((EBLK, D), lambda i: (i, 0)),
        out_shape=jax.ShapeDtypeStruct((E, D), jnp.float32),
    )(e, We)


NZ = N_PAD // 8
ZROWS_PER_TILE = NZ // 16


def _edge_compute(k_v, q_v, v_v, pe_v, dstp_v, lanes):
    perms = [(lanes ^ kk).reshape(16, 1) for kk in (1, 2, 4, 8)]
    dnums = lax.GatherDimensionNumbers(
        offset_dims=(), collapsed_slice_dims=(0,), start_index_map=(0,))

    def _shuf(x, p):
        return lax.gather(x, p, dnums, (1,),
                          mode=lax.GatherScatterMode.PROMISE_IN_BOUNDS)

    def _edge(i, _):
        z_vec = jnp.zeros((16,), jnp.float32)
        for hh in range(H):
            sl = slice(hh * DH, (hh + 1) * DH)
            sc = k_v[i, sl] * q_v[i, sl] * pe_v[i, sl]
            k_v[i, sl] = sc
            tot = sc
            for p in perms:
                tot = tot + _shuf(tot, p)
            sv = jnp.exp(jnp.minimum(jnp.maximum(tot, -5.0), 5.0))
            wv = v_v[i, sl] * sv
            v_v[i, sl] = wv
            z_vec = jnp.where(lanes == hh, sv, z_vec)
            pe_v[i, sl] = jnp.zeros((16,), jnp.float32)
        dv = dstp_v[pl.ds(i, 16)]
        colbase = (dv[0] & 7) * DH
        pe_v[i, pl.ds(colbase, DH)] = z_vec
        return 0
    lax.fori_loop(0, C, _edge, 0)


def _edge_sc_body(kh_hbm, qh_hbm, vh_hbm, pe_hbm, src_hbm, dst_hbm,
                  eo_hbm, wv_hbm, z_hbm,
                  src_a, dst_a, dstp_a, k_a, q_a, v_a,
                  src_b, dst_b, dstp_b, k_b, q_b, v_b,
                  zrow_v, pe_v,
                  shared_wv, shared_zp,
                  gk_a, gq_a, gv_a, gk_b, gq_b, gv_b,
                  se_a, sw_a, se_b, sw_b, spe, szp):
    c = lax.axis_index("c")
    s = lax.axis_index("s")
    wid = c * 16 + s
    tile_base = wid * EPT
    lanes = lax.iota(jnp.int32, 16)

    def _zero_row(i, _):
        for hh in range(H):
            k_a[i, hh * DH:(hh + 1) * DH] = jnp.zeros((16,), jnp.float32)
        return 0
    lax.fori_loop(0, C, _zero_row, 0)

    row0 = s * ROWS_PER_TILE
    def _zero_shared(j, _):
        pltpu.sync_copy(k_a, shared_wv.at[pl.ds(row0 + j * ZCHUNK, ZCHUNK)])
        return 0
    lax.fori_loop(0, ROWS_PER_TILE // ZCHUNK, _zero_shared, 0)

    zrow0 = s * ZROWS_PER_TILE
    def _zero_sharedz(j, _):
        pltpu.sync_copy(k_a, shared_zp.at[pl.ds(zrow0 + j * ZCHUNK, ZCHUNK)])
        return 0
    lax.fori_loop(0, ZROWS_PER_TILE // ZCHUNK, _zero_sharedz, 0)
    plsc.subcore_barrier()

    def _load_idx(base, src_v, dst_v, dstp_v):
        pltpu.sync_copy(src_hbm.at[pl.ds(base, C)], src_v)
        pltpu.sync_copy(dst_hbm.at[pl.ds(base, C)], dst_v)
        pltpu.sync_copy(dst_hbm.at[pl.ds(base, C)], dstp_v.at[pl.ds(0, C)])

    def _issue_gathers(src_v, dst_v, k_v, q_v, v_v, gk, gq, gv):
        pltpu.async_copy(kh_hbm.at[src_v], k_v, gk)
        pltpu.async_copy(qh_hbm.at[dst_v], q_v, gq)
        pltpu.async_copy(vh_hbm.at[src_v], v_v, gv)

    def _wait_gathers(src_v, dst_v, k_v, q_v, v_v, gk, gq, gv):
        pltpu.make_async_copy(kh_hbm.at[src_v], k_v, gk).wait()
        pltpu.make_async_copy(qh_hbm.at[dst_v], q_v, gq).wait()
        pltpu.make_async_copy(vh_hbm.at[src_v], v_v, gv).wait()

    def _zrow(dst_v):
        for g in (0, 16, 24):
            dd = dst_v[pl.ds(g, 16)]
            zrow_v[pl.ds(g, 16)] = lax.shift_right_logical(dd, 3)

    _load_idx(tile_base, src_a, dst_a, dstp_a)
    _issue_gathers(src_a, dst_a, k_a, q_a, v_a, gk_a, gq_a, gv_a)
    pltpu.async_copy(pe_hbm.at[pl.ds(tile_base, C)], pe_v, spe)

    NG = NCHUNK // 2

    def _outer(g, _):
        i0 = g * 2
        base_a = tile_base + i0 * C
        base_b = base_a + C

        @pl.when(g > 0)
        def _():
            pltpu.make_async_copy(k_b, eo_hbm.at[pl.ds(base_a - C, C)], se_b).wait()
            pltpu.make_async_copy(v_b, shared_wv.at[dst_b], sw_b).wait()

        _load_idx(base_b, src_b, dst_b, dstp_b)
        _issue_gathers(src_b, dst_b, k_b, q_b, v_b, gk_b, gq_b, gv_b)

        _wait_gathers(src_a, dst_a, k_a, q_a, v_a, gk_a, gq_a, gv_a)
        pltpu.make_async_copy(pe_hbm.at[pl.ds(base_a, C)], pe_v, spe).wait()
        _zrow(dst_a)
        _edge_compute(k_a, q_a, v_a, pe_v, dstp_a, lanes)

        pltpu.async_copy(k_a, eo_hbm.at[pl.ds(base_a, C)], se_a)
        pltpu.async_copy(v_a, shared_wv.at[dst_a], sw_a, add=True)
        pltpu.async_copy(pe_v, shared_zp.at[zrow_v], szp, add=True)
        pltpu.make_async_copy(pe_v, shared_zp.at[zrow_v], szp).wait()
        pltpu.async_copy(pe_hbm.at[pl.ds(base_b, C)], pe_v, spe)

        _wait_gathers(src_b, dst_b, k_b, q_b, v_b, gk_b, gq_b, gv_b)
        pltpu.make_async_copy(pe_hbm.at[pl.ds(base_b, C)], pe_v, spe).wait()
        _zrow(dst_b)
        _edge_compute(k_b, q_b, v_b, pe_v, dstp_b, lanes)

        pltpu.async_copy(k_b, eo_hbm.at[pl.ds(base_b, C)], se_b)
        pltpu.async_copy(v_b, shared_wv.at[dst_b], sw_b, add=True)
        pltpu.async_copy(pe_v, shared_zp.at[zrow_v], szp, add=True)

        pltpu.make_async_copy(k_a, eo_hbm.at[pl.ds(base_a, C)], se_a).wait()
        pltpu.make_async_copy(v_a, shared_wv.at[dst_a], sw_a).wait()
        pltpu.make_async_copy(pe_v, shared_zp.at[zrow_v], szp).wait()

        @pl.when(g < NG - 1)
        def _():
            _load_idx(base_b + C, src_a, dst_a, dstp_a)
            _issue_gathers(src_a, dst_a, k_a, q_a, v_a, gk_a, gq_a, gv_a)
            pltpu.async_copy(pe_hbm.at[pl.ds(base_b + C, C)], pe_v, spe)
        return 0
    lax.fori_loop(0, NG, _outer, 0)

    pltpu.make_async_copy(k_b, eo_hbm.at[pl.ds(tile_base, C)], se_b).wait()
    pltpu.make_async_copy(v_b, shared_wv.at[dst_b], sw_b).wait()

    plsc.subcore_barrier()

    def _out(j, _):
        r = row0 + j * ZCHUNK
        pltpu.sync_copy(shared_wv.at[pl.ds(r, ZCHUNK)], k_a)
        pltpu.sync_copy(k_a, wv_hbm.at[c, pl.ds(r, ZCHUNK)])
        return 0
    lax.fori_loop(0, ROWS_PER_TILE // ZCHUNK, _out, 0)

    def _outz(j, _):
        r = zrow0 + j * ZCHUNK
        pltpu.sync_copy(shared_zp.at[pl.ds(r, ZCHUNK)], k_a)
        pltpu.sync_copy(k_a, z_hbm.at[c, pl.ds(r, ZCHUNK)])
        return 0
    lax.fori_loop(0, ZROWS_PER_TILE // ZCHUNK, _outz, 0)


def _edge_sc(kh, qh, vh, pe, src, dst):
    mesh = plsc.VectorSubcoreMesh(core_axis_name="c", subcore_axis_name="s")
    iset = [
        pltpu.VMEM((C,), jnp.int32),
        pltpu.VMEM((C,), jnp.int32),
        pltpu.VMEM((C + 16,), jnp.int32),
        pltpu.VMEM((C, D), jnp.float32),
        pltpu.VMEM((C, D), jnp.float32),
        pltpu.VMEM((C, D), jnp.float32),
    ]
    f = pl.kernel(
        _edge_sc_body,
        out_type=[
            jax.ShapeDtypeStruct((E, D), jnp.float32),
            jax.ShapeDtypeStruct((2, N_PAD, D), jnp.float32),
            jax.ShapeDtypeStruct((2, NZ, D), jnp.float32),
        ],
        mesh=mesh,
        scratch_types=iset + iset + [
            pltpu.VMEM((C,), jnp.int32),
            pltpu.VMEM((C, D), jnp.float32),
            pltpu.VMEM_SHARED((N_PAD, D), jnp.float32),
            pltpu.VMEM_SHARED((NZ, D), jnp.float32),
        ] + [pltpu.SemaphoreType.DMA] * 12,
    )
    return f(kh, qh, vh, pe, src, dst)


def _node_body(wvp_ref, zp_ref, h_ref, ohw_ref, ohb_ref,
               g1_ref, b1_ref, f1w_ref, f1b_ref, f2w_ref, f2b_ref,
               g2_ref, b2_ref, out_ref):
    wv = wvp_ref[0, :N, :] + wvp_ref[1, :N, :]
    z8 = zp_ref[0, :N, :8] + zp_ref[1, :N, :8]
    rows = lax.broadcasted_iota(jnp.int32, (8, D), 0)
    cols = lax.broadcasted_iota(jnp.int32, (8, D), 1) // DH
    expand = (rows == cols).astype(jnp.float32)
    z_exp = jnp.dot(z8, expand, preferred_element_type=jnp.float32)
    h_attn = wv / (z_exp + 1e-6)
    h1 = jnp.dot(h_attn, ohw_ref[...].T, preferred_element_type=jnp.float32) + ohb_ref[...]
    h1 = h1 + h_ref[...]
    m = jnp.mean(h1, axis=0, keepdims=True)
    v = jnp.mean(h1 * h1, axis=0, keepdims=True) - m * m
    h1 = (h1 - m) * (g1_ref[...] / jnp.sqrt(v + 1e-5)) + b1_ref[...]
    t = jnp.maximum(jnp.dot(h1, f1w_ref[...].T, preferred_element_type=jnp.float32) + f1b_ref[...], 0.0)
    h2 = h1 + jnp.dot(t, f2w_ref[...].T, preferred_element_type=jnp.float32) + f2b_ref[...]
    m2 = jnp.mean(h2, axis=0, keepdims=True)
    v2 = jnp.mean(h2 * h2, axis=0, keepdims=True) - m2 * m2
    out_ref[...] = (h2 - m2) * (g2_ref[...] / jnp.sqrt(v2 + 1e-5)) + b2_ref[...]


def _node(wvp, zp, h, Oh_w, Oh_b, g1, b1, f1w, f1b, f2w, f2b, g2, b2):
    return pl.pallas_call(
        _node_body,
        out_shape=jax.ShapeDtypeStruct((N, D), jnp.float32),
    )(wvp, zp, h, Oh_w, Oh_b, g1, b1, f1w, f1b, f2w, f2b, g2, b2)


def _e1_body(eo_ref, e_ref, oew_ref, oeb_ref, out_ref, st_ref):
    x = jnp.dot(eo_ref[...], oew_ref[...].T, preferred_element_type=jnp.float32)
    x = x + oeb_ref[...] + e_ref[...]
    out_ref[...] = x
    i = pl.program_id(0)

    @pl.when(i == 0)
    def _():
        st_ref[...] = jnp.zeros_like(st_ref)

    s1 = jnp.sum(x, axis=0, keepdims=True)
    s2 = jnp.sum(x * x, axis=0, keepdims=True)
    st_ref[...] += jnp.concatenate([s1, s2], axis=0)


def _e1(e_out, e, Oe_w, Oe_b):
    return pl.pallas_call(
        _e1_body,
        grid=(EGRID,),
        in_specs=[
            pl.BlockSpec((EBLK, D), lambda i: (i, 0)),
            pl.BlockSpec((EBLK, D), lambda i: (i, 0)),
            pl.BlockSpec((D, D), lambda i: (0, 0)),
            pl.BlockSpec((1, D), lambda i: (0, 0)),
        ],
        out_specs=[
            pl.BlockSpec((EBLK, D), lambda i: (i, 0)),
            pl.BlockSpec((2, D), lambda i: (0, 0)),
        ],
        out_shape=[
            jax.ShapeDtypeStruct((E, D), jnp.float32),
            jax.ShapeDtypeStruct((2, D), jnp.float32),
        ],
    )(e_out, e, Oe_w, Oe_b)


def _e2_body(x_ref, st_ref, g_ref, b_ref, f1w_ref, f1b_ref, f2w_ref, f2b_ref,
             out_ref, st2_ref):
    m = st_ref[0:1, :] * (1.0 / E)
    v = st_ref[1:2, :] * (1.0 / E) - m * m
    a = g_ref[...] / jnp.sqrt(v + 1e-5)
    cc = b_ref[...] - m * a
    x = x_ref[...] * a + cc
    t = jnp.maximum(jnp.dot(x, f1w_ref[...].T, preferred_element_type=jnp.float32) + f1b_ref[...], 0.0)
    y = x + jnp.dot(t, f2w_ref[...].T, preferred_element_type=jnp.float32) + f2b_ref[...]
    out_ref[...] = y
    i = pl.program_id(0)

    @pl.when(i == 0)
    def _():
        st2_ref[...] = jnp.zeros_like(st2_ref)

    s1 = jnp.sum(y, axis=0, keepdims=True)
    s2 = jnp.sum(y * y, axis=0, keepdims=True)
    st2_ref[...] += jnp.concatenate([s1, s2], axis=0)


def _e2(e1_raw, st1, g, b, f1w, f1b, f2w, f2b):
    return pl.pallas_call(
        _e2_body,
        grid=(EGRID,),
        in_specs=[
            pl.BlockSpec((EBLK, D), lambda i: (i, 0)),
            pl.BlockSpec((2, D), lambda i: (0, 0)),
            pl.BlockSpec((1, D), lambda i: (0, 0)),
            pl.BlockSpec((1, D), lambda i: (0, 0)),
            pl.BlockSpec((2 * D, D), lambda i: (0, 0)),
            pl.BlockSpec((1, 2 * D), lambda i: (0, 0)),
            pl.BlockSpec((D, 2 * D), lambda i: (0, 0)),
            pl.BlockSpec((1, D), lambda i: (0, 0)),
        ],
        out_specs=[
            pl.BlockSpec((EBLK, D), lambda i: (i, 0)),
            pl.BlockSpec((2, D), lambda i: (0, 0)),
        ],
        out_shape=[
            jax.ShapeDtypeStruct((E, D), jnp.float32),
            jax.ShapeDtypeStruct((2, D), jnp.float32),
        ],
    )(e1_raw, st1, g, b, f1w, f1b, f2w, f2b)


def _e3_body(x_ref, st_ref, g_ref, b_ref, out_ref):
    m = st_ref[0:1, :] * (1.0 / E)
    v = st_ref[1:2, :] * (1.0 / E) - m * m
    a = g_ref[...] / jnp.sqrt(v + 1e-5)
    cc = b_ref[...] - m * a
    out_ref[...] = x_ref[...] * a + cc


def _e3(e2_raw, st2, g, b):
    return pl.pallas_call(
        _e3_body,
        grid=(EGRID,),
        in_specs=[
            pl.BlockSpec((EBLK, D), lambda i: (i, 0)),
            pl.BlockSpec((2, D), lambda i: (0, 0)),
            pl.BlockSpec((1, D), lambda i: (0, 0)),
            pl.BlockSpec((1, D), lambda i: (0, 0)),
        ],
        out_specs=pl.BlockSpec((EBLK, D), lambda i: (i, 0)),
        out_shape=jax.ShapeDtypeStruct((E, D), jnp.float32),
    )(e2_raw, st2, g, b)


@jax.jit
def kernel(h, e, Wq, Wk, Wv, We, Oh_w, Oh_b, Oe_w, Oe_b,
           bn1h_g, bn1h_b, bn1e_g, bn1e_b,
           fh1_w, fh1_b, fh2_w, fh2_b, fe1_w, fe1_b, fe2_w, fe2_b,
           bn2h_g, bn2h_b, bn2e_g, bn2e_b, edge_index):
    src = edge_index[0].astype(jnp.int32)
    dst = edge_index[1].astype(jnp.int32)

    qh, kh_s, vh = _qkv(h, Wq, Wk, Wv)
    pe = _pe(e, We)
    e_out, wvp, zp_packed = _edge_sc(kh_s, qh, vh, pe, src, dst)
    zp = zp_packed.reshape(2, N_PAD, 16)

    h2 = _node(wvp, zp, h, Oh_w, Oh_b.reshape(1, D),
               bn1h_g.reshape(1, D), bn1h_b.reshape(1, D),
               fh1_w, fh1_b.reshape(1, 2 * D), fh2_w, fh2_b.reshape(1, D),
               bn2h_g.reshape(1, D), bn2h_b.reshape(1, D))

    e1_raw, st1 = _e1(e_out, e, Oe_w, Oe_b.reshape(1, D))
    e2_raw, st2 = _e2(e1_raw, st1, bn1e_g.reshape(1, D), bn1e_b.reshape(1, D),
                      fe1_w, fe1_b.reshape(1, 2 * D), fe2_w, fe2_b.reshape(1, D))
    e2 = _e3(e2_raw, st2, bn2e_g.reshape(1, D), bn2e_b.reshape(1, D))
    return (h2, e2)


# speedup vs baseline: 23.8947x; 1.0705x over previous
import functools
import jax
import jax.numpy as jnp
from jax import lax
from jax.experimental import pallas as pl
from jax.experimental.pallas import tpu as pltpu
from jax.experimental.pallas import tpu_sc as plsc

N = 10000
E = 320000
D = 128
H = 8
DH = D // H

N_PAD = 10240
NTILES = 32
EPT = E // NTILES
C = 40
NCHUNK = EPT // C
ROWS_PER_TILE = N_PAD // 16
ZCHUNK = C

EBLK = 3200
EGRID = E // EBLK


def _qkv_body(h_ref, wq_ref, wk_ref, wv_ref, q_ref, k_ref, v_ref):
    hb = h_ref[...]
    q_ref[...] = jnp.dot(hb, wq_ref[...].T, preferred_element_type=jnp.float32)
    k_ref[...] = jnp.dot(hb, wk_ref[...].T, preferred_element_type=jnp.float32) * (1.0 / (DH ** 0.5))
    v_ref[...] = jnp.dot(hb, wv_ref[...].T, preferred_element_type=jnp.float32)


def _qkv(h, Wq, Wk, Wv):
    return pl.pallas_call(
        _qkv_body,
        out_shape=[jax.ShapeDtypeStruct((N, D), jnp.float32)] * 3,
    )(h, Wq, Wk, Wv)


def _pe_body(e_ref, we_ref, pe_ref):
    pe_ref[...] = jnp.dot(e_ref[...], we_ref[...].T, preferred_element_type=jnp.float32)


def _pe(e, We):
    return pl.pallas_call(
        _pe_body,
        grid=(EGRID,),
        in_specs=[
            pl.BlockSpec((EBLK, D), lambda i: (i, 0)),
            pl.BlockSpec((D, D), lambda i: (0, 0)),
        ],
        out_specs=pl.BlockSpec((EBLK, D), lambda i: (i, 0)),
        out_shape=jax.ShapeDtypeStruct((E, D), jnp.float32),
    )(e, We)


NZ = N_PAD // 8
ZROWS_PER_TILE = NZ // 16


def _edge_compute(k_v, q_v, v_v, pe_v, ix_v, lanes):
    perms = [(lanes ^ kk).reshape(16, 1) for kk in (1, 2, 4, 8)]
    dnums = lax.GatherDimensionNumbers(
        offset_dims=(), collapsed_slice_dims=(0,), start_index_map=(0,))

    def _shuf(x, p):
        return lax.gather(x, p, dnums, (1,),
                          mode=lax.GatherScatterMode.PROMISE_IN_BOUNDS)

    def _edge(i, _):
        z_vec = jnp.zeros((16,), jnp.float32)
        for hh in range(H):
            sl = slice(hh * DH, (hh + 1) * DH)
            sc = k_v[i, sl] * q_v[i, sl] * pe_v[i, sl]
            k_v[i, sl] = sc
            tot = sc
            for p in perms:
                tot = tot + _shuf(tot, p)
            sv = jnp.exp(jnp.minimum(jnp.maximum(tot, -5.0), 5.0))
            wv = v_v[i, sl] * sv
            v_v[i, sl] = wv
            z_vec = jnp.where(lanes == hh, sv, z_vec)
            pe_v[i, sl] = jnp.zeros((16,), jnp.float32)
        dv = ix_v[2, pl.ds(i, 16)]
        colbase = (dv[0] & 7) * DH
        pe_v[i, pl.ds(colbase, DH)] = z_vec
        return 0
    lax.fori_loop(0, C, _edge, 0)


def _edge_sc_body(kh_hbm, qh_hbm, vh_hbm, pe_hbm, idxp_hbm,
                  eo_hbm, wv_hbm, z_hbm,
                  ix_a, k_a, q_a, v_a,
                  ix_b, k_b, q_b, v_b,
                  zrow_v, pe_v,
                  shared_wv, shared_zp,
                  gk_a, gq_a, gv_a, gk_b, gq_b, gv_b,
                  se_a, sw_a, se_b, sw_b, spe, szp):
    c = lax.axis_index("c")
    s = lax.axis_index("s")
    wid = c * 16 + s
    tile_base = wid * EPT
    lanes = lax.iota(jnp.int32, 16)

    def _zero_row(i, _):
        for hh in range(H):
            k_a[i, hh * DH:(hh + 1) * DH] = jnp.zeros((16,), jnp.float32)
        return 0
    lax.fori_loop(0, C, _zero_row, 0)

    row0 = s * ROWS_PER_TILE
    def _zero_shared(j, _):
        pltpu.sync_copy(k_a, shared_wv.at[pl.ds(row0 + j * ZCHUNK, ZCHUNK)])
        return 0
    lax.fori_loop(0, ROWS_PER_TILE // ZCHUNK, _zero_shared, 0)

    zrow0 = s * ZROWS_PER_TILE
    def _zero_sharedz(j, _):
        pltpu.sync_copy(k_a, shared_zp.at[pl.ds(zrow0 + j * ZCHUNK, ZCHUNK)])
        return 0
    lax.fori_loop(0, ZROWS_PER_TILE // ZCHUNK, _zero_sharedz, 0)
    plsc.subcore_barrier()

    tile_chunk0 = wid * NCHUNK

    def _load_idx(cid, ix_v):
        pltpu.sync_copy(idxp_hbm.at[cid], ix_v)

    def _issue_gathers(ix_v, k_v, q_v, v_v, gk, gq, gv):
        pltpu.async_copy(kh_hbm.at[ix_v.at[0]], k_v, gk)
        pltpu.async_copy(qh_hbm.at[ix_v.at[1]], q_v, gq)
        pltpu.async_copy(vh_hbm.at[ix_v.at[0]], v_v, gv)

    def _wait_gathers(ix_v, k_v, q_v, v_v, gk, gq, gv):
        pltpu.make_async_copy(kh_hbm.at[ix_v.at[0]], k_v, gk).wait()
        pltpu.make_async_copy(qh_hbm.at[ix_v.at[1]], q_v, gq).wait()
        pltpu.make_async_copy(vh_hbm.at[ix_v.at[0]], v_v, gv).wait()

    def _zrow(ix_v):
        for g in (0, 16, 24):
            dd = ix_v[1, pl.ds(g, 16)]
            zrow_v[pl.ds(g, 16)] = lax.shift_right_logical(dd, 3)

    _load_idx(tile_chunk0, ix_a)
    _issue_gathers(ix_a, k_a, q_a, v_a, gk_a, gq_a, gv_a)
    pltpu.async_copy(pe_hbm.at[pl.ds(tile_base, C)], pe_v, spe)

    NG = NCHUNK // 2

    def _outer(g, _):
        i0 = g * 2
        base_a = tile_base + i0 * C
        base_b = base_a + C

        @pl.when(g > 0)
        def _():
            pltpu.make_async_copy(k_b, eo_hbm.at[pl.ds(base_a - C, C)], se_b).wait()
            pltpu.make_async_copy(v_b, shared_wv.at[ix_b.at[1]], sw_b).wait()

        _load_idx(tile_chunk0 + i0 + 1, ix_b)
        _issue_gathers(ix_b, k_b, q_b, v_b, gk_b, gq_b, gv_b)

        _wait_gathers(ix_a, k_a, q_a, v_a, gk_a, gq_a, gv_a)
        pltpu.make_async_copy(pe_hbm.at[pl.ds(base_a, C)], pe_v, spe).wait()
        _zrow(ix_a)
        _edge_compute(k_a, q_a, v_a, pe_v, ix_a, lanes)

        pltpu.async_copy(k_a, eo_hbm.at[pl.ds(base_a, C)], se_a)
        pltpu.async_copy(v_a, shared_wv.at[ix_a.at[1]], sw_a, add=True)
        pltpu.async_copy(pe_v, shared_zp.at[zrow_v], szp, add=True)
        pltpu.make_async_copy(pe_v, shared_zp.at[zrow_v], szp).wait()
        pltpu.async_copy(pe_hbm.at[pl.ds(base_b, C)], pe_v, spe)

        _wait_gathers(ix_b, k_b, q_b, v_b, gk_b, gq_b, gv_b)
        pltpu.make_async_copy(pe_hbm.at[pl.ds(base_b, C)], pe_v, spe).wait()
        _zrow(ix_b)
        _edge_compute(k_b, q_b, v_b, pe_v, ix_b, lanes)

        pltpu.async_copy(k_b, eo_hbm.at[pl.ds(base_b, C)], se_b)
        pltpu.async_copy(v_b, shared_wv.at[ix_b.at[1]], sw_b, add=True)
        pltpu.async_copy(pe_v, shared_zp.at[zrow_v], szp, add=True)

        pltpu.make_async_copy(k_a, eo_hbm.at[pl.ds(base_a, C)], se_a).wait()
        pltpu.make_async_copy(v_a, shared_wv.at[ix_a.at[1]], sw_a).wait()
        pltpu.make_async_copy(pe_v, shared_zp.at[zrow_v], szp).wait()

        @pl.when(g < NG - 1)
        def _():
            _load_idx(tile_chunk0 + i0 + 2, ix_a)
            _issue_gathers(ix_a, k_a, q_a, v_a, gk_a, gq_a, gv_a)
            pltpu.async_copy(pe_hbm.at[pl.ds(base_b + C, C)], pe_v, spe)
        return 0
    lax.fori_loop(0, NG, _outer, 0)

    pltpu.make_async_copy(k_b, eo_hbm.at[pl.ds(tile_base, C)], se_b).wait()
    pltpu.make_async_copy(v_b, shared_wv.at[ix_b.at[1]], sw_b).wait()

    plsc.subcore_barrier()

    def _out(j, _):
        r = row0 + j * ZCHUNK
        pltpu.sync_copy(shared_wv.at[pl.ds(r, ZCHUNK)], k_a)
        pltpu.sync_copy(k_a, wv_hbm.at[c, pl.ds(r, ZCHUNK)])
        return 0
    lax.fori_loop(0, ROWS_PER_TILE // ZCHUNK, _out, 0)

    def _outz(j, _):
        r = zrow0 + j * ZCHUNK
        pltpu.sync_copy(shared_zp.at[pl.ds(r, ZCHUNK)], k_a)
        pltpu.sync_copy(k_a, z_hbm.at[c, pl.ds(r, ZCHUNK)])
        return 0
    lax.fori_loop(0, ZROWS_PER_TILE // ZCHUNK, _outz, 0)


def _edge_sc(kh, qh, vh, pe, idxp):
    mesh = plsc.VectorSubcoreMesh(core_axis_name="c", subcore_axis_name="s")
    iset = [
        pltpu.VMEM((4, C), jnp.int32),
        pltpu.VMEM((C, D), jnp.float32),
        pltpu.VMEM((C, D), jnp.float32),
        pltpu.VMEM((C, D), jnp.float32),
    ]
    f = pl.kernel(
        _edge_sc_body,
        out_type=[
            jax.ShapeDtypeStruct((E, D), jnp.float32),
            jax.ShapeDtypeStruct((2, N_PAD, D), jnp.float32),
            jax.ShapeDtypeStruct((2, NZ, D), jnp.float32),
        ],
        mesh=mesh,
        scratch_types=iset + iset + [
            pltpu.VMEM((C,), jnp.int32),
            pltpu.VMEM((C, D), jnp.float32),
            pltpu.VMEM_SHARED((N_PAD, D), jnp.float32),
            pltpu.VMEM_SHARED((NZ, D), jnp.float32),
        ] + [pltpu.SemaphoreType.DMA] * 12,
    )
    return f(kh, qh, vh, pe, idxp)


def _node_body(wvp_ref, zp_ref, h_ref, ohw_ref, ohb_ref,
               g1_ref, b1_ref, f1w_ref, f1b_ref, f2w_ref, f2b_ref,
               g2_ref, b2_ref, out_ref):
    wv = wvp_ref[0, :N, :] + wvp_ref[1, :N, :]
    z8 = zp_ref[0, :N, :8] + zp_ref[1, :N, :8]
    rows = lax.broadcasted_iota(jnp.int32, (8, D), 0)
    cols = lax.broadcasted_iota(jnp.int32, (8, D), 1) // DH
    expand = (rows == cols).astype(jnp.float32)
    z_exp = jnp.dot(z8, expand, preferred_element_type=jnp.float32)
    h_attn = wv / (z_exp + 1e-6)
    h1 = jnp.dot(h_attn, ohw_ref[...].T, preferred_element_type=jnp.float32) + ohb_ref[...]
    h1 = h1 + h_ref[...]
    m = jnp.mean(h1, axis=0, keepdims=True)
    v = jnp.mean(h1 * h1, axis=0, keepdims=True) - m * m
    h1 = (h1 - m) * (g1_ref[...] / jnp.sqrt(v + 1e-5)) + b1_ref[...]
    t = jnp.maximum(jnp.dot(h1, f1w_ref[...].T, preferred_element_type=jnp.float32) + f1b_ref[...], 0.0)
    h2 = h1 + jnp.dot(t, f2w_ref[...].T, preferred_element_type=jnp.float32) + f2b_ref[...]
    m2 = jnp.mean(h2, axis=0, keepdims=True)
    v2 = jnp.mean(h2 * h2, axis=0, keepdims=True) - m2 * m2
    out_ref[...] = (h2 - m2) * (g2_ref[...] / jnp.sqrt(v2 + 1e-5)) + b2_ref[...]


def _node(wvp, zp, h, Oh_w, Oh_b, g1, b1, f1w, f1b, f2w, f2b, g2, b2):
    return pl.pallas_call(
        _node_body,
        out_shape=jax.ShapeDtypeStruct((N, D), jnp.float32),
    )(wvp, zp, h, Oh_w, Oh_b, g1, b1, f1w, f1b, f2w, f2b, g2, b2)


def _e1_body(eo_ref, e_ref, oew_ref, oeb_ref, out_ref, st_ref):
    x = jnp.dot(eo_ref[...], oew_ref[...].T, preferred_element_type=jnp.float32)
    x = x + oeb_ref[...] + e_ref[...]
    out_ref[...] = x
    i = pl.program_id(0)

    @pl.when(i == 0)
    def _():
        st_ref[...] = jnp.zeros_like(st_ref)

    s1 = jnp.sum(x, axis=0, keepdims=True)
    s2 = jnp.sum(x * x, axis=0, keepdims=True)
    st_ref[...] += jnp.concatenate([s1, s2], axis=0)


def _e1(e_out, e, Oe_w, Oe_b):
    return pl.pallas_call(
        _e1_body,
        grid=(EGRID,),
        in_specs=[
            pl.BlockSpec((EBLK, D), lambda i: (i, 0)),
            pl.BlockSpec((EBLK, D), lambda i: (i, 0)),
            pl.BlockSpec((D, D), lambda i: (0, 0)),
            pl.BlockSpec((1, D), lambda i: (0, 0)),
        ],
        out_specs=[
            pl.BlockSpec((EBLK, D), lambda i: (i, 0)),
            pl.BlockSpec((2, D), lambda i: (0, 0)),
        ],
        out_shape=[
            jax.ShapeDtypeStruct((E, D), jnp.float32),
            jax.ShapeDtypeStruct((2, D), jnp.float32),
        ],
    )(e_out, e, Oe_w, Oe_b)


def _e2_body(x_ref, st_ref, g_ref, b_ref, f1w_ref, f1b_ref, f2w_ref, f2b_ref,
             out_ref, st2_ref):
    m = st_ref[0:1, :] * (1.0 / E)
    v = st_ref[1:2, :] * (1.0 / E) - m * m
    a = g_ref[...] / jnp.sqrt(v + 1e-5)
    cc = b_ref[...] - m * a
    x = x_ref[...] * a + cc
    t = jnp.maximum(jnp.dot(x, f1w_ref[...].T, preferred_element_type=jnp.float32) + f1b_ref[...], 0.0)
    y = x + jnp.dot(t, f2w_ref[...].T, preferred_element_type=jnp.float32) + f2b_ref[...]
    out_ref[...] = y
    i = pl.program_id(0)

    @pl.when(i == 0)
    def _():
        st2_ref[...] = jnp.zeros_like(st2_ref)

    s1 = jnp.sum(y, axis=0, keepdims=True)
    s2 = jnp.sum(y * y, axis=0, keepdims=True)
    st2_ref[...] += jnp.concatenate([s1, s2], axis=0)


def _e2(e1_raw, st1, g, b, f1w, f1b, f2w, f2b):
    return pl.pallas_call(
        _e2_body,
        grid=(EGRID,),
        in_specs=[
            pl.BlockSpec((EBLK, D), lambda i: (i, 0)),
            pl.BlockSpec((2, D), lambda i: (0, 0)),
            pl.BlockSpec((1, D), lambda i: (0, 0)),
            pl.BlockSpec((1, D), lambda i: (0, 0)),
            pl.BlockSpec((2 * D, D), lambda i: (0, 0)),
            pl.BlockSpec((1, 2 * D), lambda i: (0, 0)),
            pl.BlockSpec((D, 2 * D), lambda i: (0, 0)),
            pl.BlockSpec((1, D), lambda i: (0, 0)),
        ],
        out_specs=[
            pl.BlockSpec((EBLK, D), lambda i: (i, 0)),
            pl.BlockSpec((2, D), lambda i: (0, 0)),
        ],
        out_shape=[
            jax.ShapeDtypeStruct((E, D), jnp.float32),
            jax.ShapeDtypeStruct((2, D), jnp.float32),
        ],
    )(e1_raw, st1, g, b, f1w, f1b, f2w, f2b)


def _e3_body(x_ref, st_ref, g_ref, b_ref, out_ref):
    m = st_ref[0:1, :] * (1.0 / E)
    v = st_ref[1:2, :] * (1.0 / E) - m * m
    a = g_ref[...] / jnp.sqrt(v + 1e-5)
    cc = b_ref[...] - m * a
    out_ref[...] = x_ref[...] * a + cc


def _e3(e2_raw, st2, g, b):
    return pl.pallas_call(
        _e3_body,
        grid=(EGRID,),
        in_specs=[
            pl.BlockSpec((EBLK, D), lambda i: (i, 0)),
            pl.BlockSpec((2, D), lambda i: (0, 0)),
            pl.BlockSpec((1, D), lambda i: (0, 0)),
            pl.BlockSpec((1, D), lambda i: (0, 0)),
        ],
        out_specs=pl.BlockSpec((EBLK, D), lambda i: (i, 0)),
        out_shape=jax.ShapeDtypeStruct((E, D), jnp.float32),
    )(e2_raw, st2, g, b)


@jax.jit
def kernel(h, e, Wq, Wk, Wv, We, Oh_w, Oh_b, Oe_w, Oe_b,
           bn1h_g, bn1h_b, bn1e_g, bn1e_b,
           fh1_w, fh1_b, fh2_w, fh2_b, fe1_w, fe1_b, fe2_w, fe2_b,
           bn2h_g, bn2h_b, bn2e_g, bn2e_b, edge_index):
    src = edge_index[0].astype(jnp.int32)
    dst = edge_index[1].astype(jnp.int32)
    sr = src.reshape(E // C, 1, C)
    dr = dst.reshape(E // C, 1, C)
    idxp = jnp.concatenate([sr, dr, dr, dr], axis=1)

    qh, kh_s, vh = _qkv(h, Wq, Wk, Wv)
    pe = _pe(e, We)
    e_out, wvp, zp_packed = _edge_sc(kh_s, qh, vh, pe, idxp)
    zp = zp_packed.reshape(2, N_PAD, 16)

    h2 = _node(wvp, zp, h, Oh_w, Oh_b.reshape(1, D),
               bn1h_g.reshape(1, D), bn1h_b.reshape(1, D),
               fh1_w, fh1_b.reshape(1, 2 * D), fh2_w, fh2_b.reshape(1, D),
               bn2h_g.reshape(1, D), bn2h_b.reshape(1, D))

    e1_raw, st1 = _e1(e_out, e, Oe_w, Oe_b.reshape(1, D))
    e2_raw, st2 = _e2(e1_raw, st1, bn1e_g.reshape(1, D), bn1e_b.reshape(1, D),
                      fe1_w, fe1_b.reshape(1, 2 * D), fe2_w, fe2_b.reshape(1, D))
    e2 = _e3(e2_raw, st2, bn2e_g.reshape(1, D), bn2e_b.reshape(1, D))
    return (h2, e2)


# speedup vs baseline: 25.4522x; 1.0652x over previous
import functools
import jax
import jax.numpy as jnp
from jax import lax
from jax.experimental import pallas as pl
from jax.experimental.pallas import tpu as pltpu
from jax.experimental.pallas import tpu_sc as plsc

N = 10000
E = 320000
D = 128
H = 8
DH = D // H

N_PAD = 10240
NTILES = 32
EPT = E // NTILES
C = 40
NCHUNK = EPT // C
ROWS_PER_TILE = N_PAD // 16
ZCHUNK = C

EBLK = 3200
EGRID = E // EBLK


def _qkv_body(h_ref, wq_ref, wk_ref, wv_ref, q_ref, k_ref, v_ref):
    hb = h_ref[...]
    q_ref[...] = jnp.dot(hb, wq_ref[...].T, preferred_element_type=jnp.float32)
    k_ref[...] = jnp.dot(hb, wk_ref[...].T, preferred_element_type=jnp.float32) * (1.0 / (DH ** 0.5))
    v_ref[...] = jnp.dot(hb, wv_ref[...].T, preferred_element_type=jnp.float32)


def _qkv(h, Wq, Wk, Wv):
    return pl.pallas_call(
        _qkv_body,
        out_shape=[jax.ShapeDtypeStruct((N, D), jnp.float32)] * 3,
    )(h, Wq, Wk, Wv)


def _pe_body(e_ref, we_ref, pe_ref):
    pe_ref[...] = jnp.dot(e_ref[...], we_ref[...].T, preferred_element_type=jnp.float32)


def _pe(e, We):
    return pl.pallas_call(
        _pe_body,
        grid=(EGRID,),
        in_specs=[
            pl.BlockSpec((EBLK, D), lambda i: (i, 0)),
            pl.BlockSpec((D, D), lambda i: (0, 0)),
        ],
        out_specs=pl.BlockSpec((EBLK, D), lambda i: (i, 0)),
        out_shape=jax.ShapeDtypeStruct((E, D), jnp.float32),
    )(e, We)


NZ = N_PAD // 8
ZROWS_PER_TILE = NZ // 16


def _edge_compute(k_v, q_v, v_v, pe_v, ix_v, lanes):
    perms = [(lanes ^ kk).reshape(16, 1) for kk in (1, 2, 4, 8)]
    dnums = lax.GatherDimensionNumbers(
        offset_dims=(), collapsed_slice_dims=(0,), start_index_map=(0,))

    def _shuf(x, p):
        return lax.gather(x, p, dnums, (1,),
                          mode=lax.GatherScatterMode.PROMISE_IN_BOUNDS)

    def _edge(i, _):
        z_vec = jnp.zeros((16,), jnp.float32)
        for hh in range(H):
            sl = slice(hh * DH, (hh + 1) * DH)
            sc = k_v[i, sl] * q_v[i, sl] * pe_v[i, sl]
            k_v[i, sl] = sc
            tot = sc
            for p in perms:
                tot = tot + _shuf(tot, p)
            sv = jnp.exp(jnp.minimum(jnp.maximum(tot, -5.0), 5.0))
            wv = v_v[i, sl] * sv
            v_v[i, sl] = wv
            z_vec = jnp.where(lanes == hh, sv, z_vec)
            q_v[i, sl] = jnp.zeros((16,), jnp.float32)
        dv = ix_v[2, pl.ds(i, 16)]
        colbase = (dv[0] & 7) * DH
        q_v[i, pl.ds(colbase, DH)] = z_vec
        return 0
    lax.fori_loop(0, C, _edge, 0)


def _edge_sc_body(kh_hbm, qh_hbm, vh_hbm, pe_hbm, idxp_hbm,
                  eo_hbm, wv_hbm, z_hbm,
                  ix_a, k_a, q_a, v_a,
                  ix_b, k_b, q_b, v_b,
                  zrow_a, zrow_b, pe_v,
                  shared_wv, shared_zp,
                  gk_a, gq_a, gv_a, gk_b, gq_b, gv_b,
                  se_a, sw_a, sz_a, se_b, sw_b, sz_b, spe):
    c = lax.axis_index("c")
    s = lax.axis_index("s")
    wid = c * 16 + s
    tile_base = wid * EPT
    lanes = lax.iota(jnp.int32, 16)

    def _zero_row(i, _):
        for hh in range(H):
            k_a[i, hh * DH:(hh + 1) * DH] = jnp.zeros((16,), jnp.float32)
        return 0
    lax.fori_loop(0, C, _zero_row, 0)

    row0 = s * ROWS_PER_TILE
    def _zero_shared(j, _):
        pltpu.sync_copy(k_a, shared_wv.at[pl.ds(row0 + j * ZCHUNK, ZCHUNK)])
        return 0
    lax.fori_loop(0, ROWS_PER_TILE // ZCHUNK, _zero_shared, 0)

    zrow0 = s * ZROWS_PER_TILE
    def _zero_sharedz(j, _):
        pltpu.sync_copy(k_a, shared_zp.at[pl.ds(zrow0 + j * ZCHUNK, ZCHUNK)])
        return 0
    lax.fori_loop(0, ZROWS_PER_TILE // ZCHUNK, _zero_sharedz, 0)
    plsc.subcore_barrier()

    tile_chunk0 = wid * NCHUNK

    def _load_idx(cid, ix_v):
        pltpu.sync_copy(idxp_hbm.at[cid], ix_v)

    def _issue_gathers(ix_v, k_v, q_v, v_v, gk, gq, gv):
        pltpu.async_copy(kh_hbm.at[ix_v.at[0]], k_v, gk)
        pltpu.async_copy(qh_hbm.at[ix_v.at[1]], q_v, gq)
        pltpu.async_copy(vh_hbm.at[ix_v.at[0]], v_v, gv)

    def _wait_gathers(ix_v, k_v, q_v, v_v, gk, gq, gv):
        pltpu.make_async_copy(kh_hbm.at[ix_v.at[0]], k_v, gk).wait()
        pltpu.make_async_copy(qh_hbm.at[ix_v.at[1]], q_v, gq).wait()
        pltpu.make_async_copy(vh_hbm.at[ix_v.at[0]], v_v, gv).wait()

    def _zrow(ix_v, zrow_v):
        for g in (0, 16, 24):
            dd = ix_v[1, pl.ds(g, 16)]
            zrow_v[pl.ds(g, 16)] = lax.shift_right_logical(dd, 3)

    _load_idx(tile_chunk0, ix_a)
    _issue_gathers(ix_a, k_a, q_a, v_a, gk_a, gq_a, gv_a)
    pltpu.async_copy(pe_hbm.at[pl.ds(tile_base, C)], pe_v, spe)

    NG = NCHUNK // 2

    def _outer(g, _):
        i0 = g * 2
        base_a = tile_base + i0 * C
        base_b = base_a + C

        @pl.when(g > 0)
        def _():
            pltpu.make_async_copy(k_b, eo_hbm.at[pl.ds(base_a - C, C)], se_b).wait()
            pltpu.make_async_copy(v_b, shared_wv.at[ix_b.at[1]], sw_b).wait()
            pltpu.make_async_copy(q_b, shared_zp.at[zrow_b], sz_b).wait()

        _load_idx(tile_chunk0 + i0 + 1, ix_b)
        _issue_gathers(ix_b, k_b, q_b, v_b, gk_b, gq_b, gv_b)

        _wait_gathers(ix_a, k_a, q_a, v_a, gk_a, gq_a, gv_a)
        pltpu.make_async_copy(pe_hbm.at[pl.ds(base_a, C)], pe_v, spe).wait()
        _zrow(ix_a, zrow_a)
        _edge_compute(k_a, q_a, v_a, pe_v, ix_a, lanes)

        pltpu.async_copy(pe_hbm.at[pl.ds(base_b, C)], pe_v, spe)
        pltpu.async_copy(k_a, eo_hbm.at[pl.ds(base_a, C)], se_a)
        pltpu.async_copy(v_a, shared_wv.at[ix_a.at[1]], sw_a, add=True)
        pltpu.async_copy(q_a, shared_zp.at[zrow_a], sz_a, add=True)

        _wait_gathers(ix_b, k_b, q_b, v_b, gk_b, gq_b, gv_b)
        pltpu.make_async_copy(pe_hbm.at[pl.ds(base_b, C)], pe_v, spe).wait()
        _zrow(ix_b, zrow_b)
        _edge_compute(k_b, q_b, v_b, pe_v, ix_b, lanes)

        pltpu.async_copy(k_b, eo_hbm.at[pl.ds(base_b, C)], se_b)
        pltpu.async_copy(v_b, shared_wv.at[ix_b.at[1]], sw_b, add=True)
        pltpu.async_copy(q_b, shared_zp.at[zrow_b], sz_b, add=True)

        pltpu.make_async_copy(k_a, eo_hbm.at[pl.ds(base_a, C)], se_a).wait()
        pltpu.make_async_copy(v_a, shared_wv.at[ix_a.at[1]], sw_a).wait()
        pltpu.make_async_copy(q_a, shared_zp.at[zrow_a], sz_a).wait()

        @pl.when(g < NG - 1)
        def _():
            _load_idx(tile_chunk0 + i0 + 2, ix_a)
            _issue_gathers(ix_a, k_a, q_a, v_a, gk_a, gq_a, gv_a)
            pltpu.async_copy(pe_hbm.at[pl.ds(base_b + C, C)], pe_v, spe)
        return 0
    lax.fori_loop(0, NG, _outer, 0)

    pltpu.make_async_copy(k_b, eo_hbm.at[pl.ds(tile_base, C)], se_b).wait()
    pltpu.make_async_copy(v_b, shared_wv.at[ix_b.at[1]], sw_b).wait()
    pltpu.make_async_copy(q_b, shared_zp.at[zrow_b], sz_b).wait()

    plsc.subcore_barrier()

    def _out(j, _):
        r = row0 + j * ZCHUNK
        pltpu.sync_copy(shared_wv.at[pl.ds(r, ZCHUNK)], k_a)
        pltpu.sync_copy(k_a, wv_hbm.at[c, pl.ds(r, ZCHUNK)])
        return 0
    lax.fori_loop(0, ROWS_PER_TILE // ZCHUNK, _out, 0)

    def _outz(j, _):
        r = zrow0 + j * ZCHUNK
        pltpu.sync_copy(shared_zp.at[pl.ds(r, ZCHUNK)], k_a)
        pltpu.sync_copy(k_a, z_hbm.at[c, pl.ds(r, ZCHUNK)])
        return 0
    lax.fori_loop(0, ZROWS_PER_TILE // ZCHUNK, _outz, 0)


def _edge_sc(kh, qh, vh, pe, idxp):
    mesh = plsc.VectorSubcoreMesh(core_axis_name="c", subcore_axis_name="s")
    iset = [
        pltpu.VMEM((4, C), jnp.int32),
        pltpu.VMEM((C, D), jnp.float32),
        pltpu.VMEM((C, D), jnp.float32),
        pltpu.VMEM((C, D), jnp.float32),
    ]
    f = pl.kernel(
        _edge_sc_body,
        out_type=[
            jax.ShapeDtypeStruct((E, D), jnp.float32),
            jax.ShapeDtypeStruct((2, N_PAD, D), jnp.float32),
            jax.ShapeDtypeStruct((2, NZ, D), jnp.float32),
        ],
        mesh=mesh,
        scratch_types=iset + iset + [
            pltpu.VMEM((C,), jnp.int32),
            pltpu.VMEM((C,), jnp.int32),
            pltpu.VMEM((C, D), jnp.float32),
            pltpu.VMEM_SHARED((N_PAD, D), jnp.float32),
            pltpu.VMEM_SHARED((NZ, D), jnp.float32),
        ] + [pltpu.SemaphoreType.DMA] * 13,
    )
    return f(kh, qh, vh, pe, idxp)


def _node_body(wvp_ref, zp_ref, h_ref, ohw_ref, ohb_ref,
               g1_ref, b1_ref, f1w_ref, f1b_ref, f2w_ref, f2b_ref,
               g2_ref, b2_ref, out_ref):
    wv = wvp_ref[0, :N, :] + wvp_ref[1, :N, :]
    z8 = zp_ref[0, :N, :8] + zp_ref[1, :N, :8]
    rows = lax.broadcasted_iota(jnp.int32, (8, D), 0)
    cols = lax.broadcasted_iota(jnp.int32, (8, D), 1) // DH
    expand = (rows == cols).astype(jnp.float32)
    z_exp = jnp.dot(z8, expand, preferred_element_type=jnp.float32)
    h_attn = wv / (z_exp + 1e-6)
    h1 = jnp.dot(h_attn, ohw_ref[...].T, preferred_element_type=jnp.float32) + ohb_ref[...]
    h1 = h1 + h_ref[...]
    m = jnp.mean(h1, axis=0, keepdims=True)
    v = jnp.mean(h1 * h1, axis=0, keepdims=True) - m * m
    h1 = (h1 - m) * (g1_ref[...] / jnp.sqrt(v + 1e-5)) + b1_ref[...]
    t = jnp.maximum(jnp.dot(h1, f1w_ref[...].T, preferred_element_type=jnp.float32) + f1b_ref[...], 0.0)
    h2 = h1 + jnp.dot(t, f2w_ref[...].T, preferred_element_type=jnp.float32) + f2b_ref[...]
    m2 = jnp.mean(h2, axis=0, keepdims=True)
    v2 = jnp.mean(h2 * h2, axis=0, keepdims=True) - m2 * m2
    out_ref[...] = (h2 - m2) * (g2_ref[...] / jnp.sqrt(v2 + 1e-5)) + b2_ref[...]


def _node(wvp, zp, h, Oh_w, Oh_b, g1, b1, f1w, f1b, f2w, f2b, g2, b2):
    return pl.pallas_call(
        _node_body,
        out_shape=jax.ShapeDtypeStruct((N, D), jnp.float32),
    )(wvp, zp, h, Oh_w, Oh_b, g1, b1, f1w, f1b, f2w, f2b, g2, b2)


def _e1_body(eo_ref, e_ref, oew_ref, oeb_ref, out_ref, st_ref):
    x = jnp.dot(eo_ref[...], oew_ref[...].T, preferred_element_type=jnp.float32)
    x = x + oeb_ref[...] + e_ref[...]
    out_ref[...] = x
    i = pl.program_id(0)

    @pl.when(i == 0)
    def _():
        st_ref[...] = jnp.zeros_like(st_ref)

    s1 = jnp.sum(x, axis=0, keepdims=True)
    s2 = jnp.sum(x * x, axis=0, keepdims=True)
    st_ref[...] += jnp.concatenate([s1, s2], axis=0)


def _e1(e_out, e, Oe_w, Oe_b):
    return pl.pallas_call(
        _e1_body,
        grid=(EGRID,),
        in_specs=[
            pl.BlockSpec((EBLK, D), lambda i: (i, 0)),
            pl.BlockSpec((EBLK, D), lambda i: (i, 0)),
            pl.BlockSpec((D, D), lambda i: (0, 0)),
            pl.BlockSpec((1, D), lambda i: (0, 0)),
        ],
        out_specs=[
            pl.BlockSpec((EBLK, D), lambda i: (i, 0)),
            pl.BlockSpec((2, D), lambda i: (0, 0)),
        ],
        out_shape=[
            jax.ShapeDtypeStruct((E, D), jnp.float32),
            jax.ShapeDtypeStruct((2, D), jnp.float32),
        ],
    )(e_out, e, Oe_w, Oe_b)


def _e2_body(x_ref, st_ref, g_ref, b_ref, f1w_ref, f1b_ref, f2w_ref, f2b_ref,
             out_ref, st2_ref):
    m = st_ref[0:1, :] * (1.0 / E)
    v = st_ref[1:2, :] * (1.0 / E) - m * m
    a = g_ref[...] / jnp.sqrt(v + 1e-5)
    cc = b_ref[...] - m * a
    x = x_ref[...] * a + cc
    t = jnp.maximum(jnp.dot(x, f1w_ref[...].T, preferred_element_type=jnp.float32) + f1b_ref[...], 0.0)
    y = x + jnp.dot(t, f2w_ref[...].T, preferred_element_type=jnp.float32) + f2b_ref[...]
    out_ref[...] = y
    i = pl.program_id(0)

    @pl.when(i == 0)
    def _():
        st2_ref[...] = jnp.zeros_like(st2_ref)

    s1 = jnp.sum(y, axis=0, keepdims=True)
    s2 = jnp.sum(y * y, axis=0, keepdims=True)
    st2_ref[...] += jnp.concatenate([s1, s2], axis=0)


def _e2(e1_raw, st1, g, b, f1w, f1b, f2w, f2b):
    return pl.pallas_call(
        _e2_body,
        grid=(EGRID,),
        in_specs=[
            pl.BlockSpec((EBLK, D), lambda i: (i, 0)),
            pl.BlockSpec((2, D), lambda i: (0, 0)),
            pl.BlockSpec((1, D), lambda i: (0, 0)),
            pl.BlockSpec((1, D), lambda i: (0, 0)),
            pl.BlockSpec((2 * D, D), lambda i: (0, 0)),
            pl.BlockSpec((1, 2 * D), lambda i: (0, 0)),
            pl.BlockSpec((D, 2 * D), lambda i: (0, 0)),
            pl.BlockSpec((1, D), lambda i: (0, 0)),
        ],
        out_specs=[
            pl.BlockSpec((EBLK, D), lambda i: (i, 0)),
            pl.BlockSpec((2, D), lambda i: (0, 0)),
        ],
        out_shape=[
            jax.ShapeDtypeStruct((E, D), jnp.float32),
            jax.ShapeDtypeStruct((2, D), jnp.float32),
        ],
    )(e1_raw, st1, g, b, f1w, f1b, f2w, f2b)


def _e3_body(x_ref, st_ref, g_ref, b_ref, out_ref):
    m = st_ref[0:1, :] * (1.0 / E)
    v = st_ref[1:2, :] * (1.0 / E) - m * m
    a = g_ref[...] / jnp.sqrt(v + 1e-5)
    cc = b_ref[...] - m * a
    out_ref[...] = x_ref[...] * a + cc


def _e3(e2_raw, st2, g, b):
    return pl.pallas_call(
        _e3_body,
        grid=(EGRID,),
        in_specs=[
            pl.BlockSpec((EBLK, D), lambda i: (i, 0)),
            pl.BlockSpec((2, D), lambda i: (0, 0)),
            pl.BlockSpec((1, D), lambda i: (0, 0)),
            pl.BlockSpec((1, D), lambda i: (0, 0)),
        ],
        out_specs=pl.BlockSpec((EBLK, D), lambda i: (i, 0)),
        out_shape=jax.ShapeDtypeStruct((E, D), jnp.float32),
    )(e2_raw, st2, g, b)


@jax.jit
def kernel(h, e, Wq, Wk, Wv, We, Oh_w, Oh_b, Oe_w, Oe_b,
           bn1h_g, bn1h_b, bn1e_g, bn1e_b,
           fh1_w, fh1_b, fh2_w, fh2_b, fe1_w, fe1_b, fe2_w, fe2_b,
           bn2h_g, bn2h_b, bn2e_g, bn2e_b, edge_index):
    src = edge_index[0].astype(jnp.int32)
    dst = edge_index[1].astype(jnp.int32)
    sr = src.reshape(E // C, 1, C)
    dr = dst.reshape(E // C, 1, C)
    idxp = jnp.concatenate([sr, dr, dr, dr], axis=1)

    qh, kh_s, vh = _qkv(h, Wq, Wk, Wv)
    pe = _pe(e, We)
    e_out, wvp, zp_packed = _edge_sc(kh_s, qh, vh, pe, idxp)
    zp = zp_packed.reshape(2, N_PAD, 16)

    h2 = _node(wvp, zp, h, Oh_w, Oh_b.reshape(1, D),
               bn1h_g.reshape(1, D), bn1h_b.reshape(1, D),
               fh1_w, fh1_b.reshape(1, 2 * D), fh2_w, fh2_b.reshape(1, D),
               bn2h_g.reshape(1, D), bn2h_b.reshape(1, D))

    e1_raw, st1 = _e1(e_out, e, Oe_w, Oe_b.reshape(1, D))
    e2_raw, st2 = _e2(e1_raw, st1, bn1e_g.reshape(1, D), bn1e_b.reshape(1, D),
                      fe1_w, fe1_b.reshape(1, 2 * D), fe2_w, fe2_b.reshape(1, D))
    e2 = _e3(e2_raw, st2, bn2e_g.reshape(1, D), bn2e_b.reshape(1, D))
    return (h2, e2)
